# Initial kernel scaffold; baseline (speedup 1.0000x reference)
import jax
import jax.numpy as jnp
from jax.experimental import pallas as pl


def kernel(xyz, points, W0, b0, gamma0, beta0, W1, b1, gamma1, beta1, W2, b2, gamma2, beta2):
    raise NotImplementedError("write your pallas kernel here")


# speedup vs baseline: 3.1359x; 3.1359x over previous
import functools
import jax
import jax.numpy as jnp
import numpy as np
from jax.experimental import pallas as pl
from jax.experimental.pallas import tpu as pltpu

B, N, D = 4, 8192, 32
NPOINT, NSAMPLE = 2048, 32
EPS = 1e-5

TS = 256
BIGF = 3.0e38


def _knn_body(q_ref, p_ref, idx_ref):
    q = q_ref[0]
    p = p_ref[0]
    dg = jax.lax.dot_general(q, p, (((1,), (1,)), ((), ())),
                             preferred_element_type=jnp.float32)
    q2 = jnp.sum(q * q, axis=1, keepdims=True)
    p2 = jnp.sum(p * p, axis=1)[None, :]
    d = (-2.0 * dg + q2) + p2
    liota = jax.lax.broadcasted_iota(jnp.int32, (TS, N), 1)
    bigi = jnp.int32(N)
    cols = []
    for j in range(NSAMPLE):
        m = jnp.min(d, axis=1)
        amin = jnp.min(jnp.where(d <= m[:, None], liota, bigi), axis=1)
        cols.append(amin)
        d = jnp.where(liota == amin[:, None], BIGF, d)
    idx_ref[0] = jnp.stack(cols, axis=1)


def _knn(new_points, points):
    return pl.pallas_call(
        _knn_body,
        grid=(B, NPOINT // TS),
        in_specs=[
            pl.BlockSpec((1, TS, D), lambda b, s: (b, s, 0)),
            pl.BlockSpec((1, N, D), lambda b, s: (b, 0, 0)),
        ],
        out_specs=pl.BlockSpec((1, TS, NSAMPLE), lambda b, s: (b, s, 0)),
        out_shape=jax.ShapeDtypeStruct((B, NPOINT, NSAMPLE), jnp.int32),
    )(new_points, points)


MT = 8192


def _layer_body(nsteps, cin, cout, first, x_ref, w_ref, bb_ref, st_ref,
                y_ref, acc_ref):
    g = pl.program_id(0)
    x = x_ref[...]
    if not first:
        mu = st_ref[0, :cin][None, :]
        inv = st_ref[1, :cin][None, :]
        gm = st_ref[2, :cin][None, :]
        be = st_ref[3, :cin][None, :]
        x = jnp.maximum((x - mu) * inv * gm + be, 0.0)
    w = w_ref[...]
    y = jax.lax.dot_general(x, w, (((1,), (1,)), ((), ())),
                            preferred_element_type=jnp.float32)
    y = y + bb_ref[0, :cout][None, :]
    y_ref[...] = y
    s1 = jnp.sum(y, axis=0)
    s2 = jnp.sum(y * y, axis=0)
    part = jnp.concatenate([s1[None, :], s2[None, :],
                            jnp.zeros((6, cout), jnp.float32)], axis=0)

    @pl.when(g == 0)
    def _init():
        acc_ref[...] = jnp.zeros_like(acc_ref)

    acc_ref[...] += part


def _layer(x, w, bvec, stats, first):
    m, cin = x.shape
    cout = w.shape[0]
    nsteps = m // MT
    body = functools.partial(_layer_body, nsteps, cin, cout, first)
    bb = jnp.broadcast_to(bvec[None, :], (8, cout))
    y, acc = pl.pallas_call(
        body,
        grid=(nsteps,),
        in_specs=[
            pl.BlockSpec((MT, cin), lambda g: (g, 0)),
            pl.BlockSpec((cout, cin), lambda g: (0, 0)),
            pl.BlockSpec((8, cout), lambda g: (0, 0)),
            pl.BlockSpec((4, cin), lambda g: (0, 0)),
        ],
        out_specs=[
            pl.BlockSpec((MT, cout), lambda g: (g, 0)),
            pl.BlockSpec((8, cout), lambda g: (0, 0)),
        ],
        out_shape=[
            jax.ShapeDtypeStruct((m, cout), jnp.float32),
            jax.ShapeDtypeStruct((8, cout), jnp.float32),
        ],
    )(x, w, bb, stats)
    return y, acc


def _final_body(cin, st_ref, x_ref, o_ref):
    mu = st_ref[0, :cin][None, None, :]
    inv = st_ref[1, :cin][None, None, :]
    gm = st_ref[2, :cin][None, None, :]
    be = st_ref[3, :cin][None, None, :]
    x = x_ref[...]
    x = jnp.maximum((x - mu) * inv * gm + be, 0.0)
    o_ref[...] = jnp.max(x, axis=1)


def _final(x3, stats):
    rows, k, cin = x3.shape
    ts2 = 256
    body = functools.partial(_final_body, cin)
    return pl.pallas_call(
        body,
        grid=(rows // ts2,),
        in_specs=[
            pl.BlockSpec((4, cin), lambda g: (0, 0)),
            pl.BlockSpec((ts2, k, cin), lambda g: (g, 0, 0)),
        ],
        out_specs=pl.BlockSpec((ts2, cin), lambda g: (g, 0)),
        out_shape=jax.ShapeDtypeStruct((rows, cin), jnp.float32),
    )(stats, x3)


def _stats_from_acc(acc, m, g, be):
    s1 = acc[0]
    s2 = acc[1]
    mu = s1 / m
    var = s2 / m - mu * mu
    inv = 1.0 / jnp.sqrt(var + EPS)
    return jnp.stack([mu, inv, g, be], axis=0)


def kernel(xyz, points, W0, b0, gamma0, beta0, W1, b1, gamma1, beta1,
           W2, b2, gamma2, beta2):
    idx_perm = jax.random.permutation(jax.random.key(42), N)[:NPOINT]
    new_xyz = xyz[:, idx_perm, :]
    new_points = points[:, idx_perm, :]

    group_idx = _knn(new_points, points)

    bidx = jnp.arange(B)[:, None, None]
    grouped = points[bidx, group_idx]

    m = B * NPOINT * NSAMPLE
    x0 = grouped.reshape(m, D)
    dummy = jnp.zeros((4, D), jnp.float32)
    y0, acc0 = _layer(x0, W0, b0, dummy, first=True)
    st0 = _stats_from_acc(acc0, m, gamma0, beta0)
    y1, acc1 = _layer(y0, W1, b1, st0, first=False)
    st1 = _stats_from_acc(acc1, m, gamma1, beta1)
    y2, acc2 = _layer(y1, W2, b2, st1, first=False)
    st2 = _stats_from_acc(acc2, m, gamma2, beta2)

    x3 = y2.reshape(B * NPOINT, NSAMPLE, W2.shape[0])
    out = _final(x3, st2)
    return (new_xyz, out.reshape(B, NPOINT, W2.shape[0]))
